# baseline (speedup 1.0000x reference)
import functools

import jax
import jax.numpy as jnp
from jax import lax
from jax.experimental import pallas as pl
from jax.experimental.pallas import tpu as pltpu
from jax.experimental.pallas import tpu_sc as plsc

_EPS = 1e-12

_SC_CORES = 2
_SC_SUBCORES = 16
_SC_WORKERS = _SC_CORES * _SC_SUBCORES
_IDX_CHUNK = 128


def _emb_prep_body(emb_n_ref, w_out_t_ref, b_out_ref, embw_ref):
    embw_ref[...] = (
        lax.dot_general(emb_n_ref[...], w_out_t_ref[...], (((1,), (0,)), ((), ())),
                        preferred_element_type=jnp.float32)
        + b_out_ref[...]
    )


def _scores_argmin_body(zfn_ref, emb_n_ref, idx_ref):
    s = lax.dot_general(zfn_ref[...], emb_n_ref[...], (((1,), (1,)), ((), ())),
                        preferred_element_type=jnp.float32)
    n_e = s.shape[1]
    rows = s.shape[0]
    out = []
    for r0 in range(0, rows, 128):
        sr = s[r0:r0 + 128, :]
        m = sr[:, 0:128]
        cidx = jnp.zeros((128, 128), jnp.int32)
        for c in range(1, n_e // 128):
            sv = sr[:, c * 128:(c + 1) * 128]
            upd = sv > m
            m = jnp.where(upd, sv, m)
            cidx = jnp.where(upd, jnp.int32(c), cidx)
        gidx = cidx * 128 + lax.broadcasted_iota(jnp.int32, (128, 128), 1)
        mx = jnp.max(m, axis=1, keepdims=True)
        cand = jnp.where(m == mx, gidx, n_e)
        out.append(jnp.min(cand, axis=1))
    idx_ref[0, 0, :] = jnp.concatenate(out, axis=0)


def _sc_gather(n_tok, n_e, d):
    chunks_per_worker = n_tok // (_SC_WORKERS * _IDX_CHUNK)
    mesh = plsc.VectorSubcoreMesh(core_axis_name="c", subcore_axis_name="s")

    @functools.partial(
        pl.kernel,
        mesh=mesh,
        out_type=jax.ShapeDtypeStruct((n_tok, d), jnp.float32),
        scratch_types=[
            pltpu.VMEM((chunks_per_worker, _IDX_CHUNK), jnp.int32),
            pltpu.VMEM((_IDX_CHUNK, d), jnp.float32),
            pltpu.VMEM((_IDX_CHUNK, d), jnp.float32),
            pltpu.SemaphoreType.DMA,
        ],
    )
    def gather(table_hbm, idx_hbm, out_hbm, idx_v, rows0_v, rows1_v, sem):
        wid = lax.axis_index("s") * _SC_CORES + lax.axis_index("c")
        first = wid * chunks_per_worker
        pltpu.sync_copy(idx_hbm.at[pl.ds(first, chunks_per_worker)], idx_v)
        rows = [rows0_v, rows1_v]
        copies = [
            pltpu.async_copy(table_hbm.at[idx_v.at[j]], rows[j % 2], sem)
            for j in range(chunks_per_worker)
        ]
        for j in range(chunks_per_worker):
            copies[j].wait()
            pltpu.sync_copy(
                rows[j % 2],
                out_hbm.at[pl.ds((first + j) * _IDX_CHUNK, _IDX_CHUNK)])

    return gather


def kernel(z, mask, W_in, b_in, W_out, b_out, emb):
    b, s, latent = z.shape
    n_tok = b * s
    n_e, e_dim = emb.shape

    cb_blk = 1024
    tok_blk = 256

    zf = z.reshape(n_tok, latent) @ W_in.T + b_in
    nz = jnp.linalg.norm(zf, axis=-1, keepdims=True)
    zfn = zf / jnp.maximum(nz, _EPS)
    ne = jnp.linalg.norm(emb, axis=-1, keepdims=True)
    emb_n = emb / jnp.maximum(ne, _EPS)
    zfn_b = zfn.astype(jnp.bfloat16)
    emb_nb = emb_n.astype(jnp.bfloat16)

    embw = pl.pallas_call(
        _emb_prep_body,
        grid=(n_e // cb_blk,),
        in_specs=[
            pl.BlockSpec((cb_blk, e_dim), lambda i: (i, 0)),
            pl.BlockSpec((e_dim, latent), lambda i: (0, 0)),
            pl.BlockSpec((1, latent), lambda i: (0, 0)),
        ],
        out_specs=pl.BlockSpec((cb_blk, latent), lambda i: (i, 0)),
        out_shape=jax.ShapeDtypeStruct((n_e, latent), jnp.float32),
    )(emb_n, W_out.T, b_out.reshape(1, latent))

    idx3 = pl.pallas_call(
        _scores_argmin_body,
        grid=(n_tok // tok_blk,),
        in_specs=[
            pl.BlockSpec((tok_blk, e_dim), lambda i: (i, 0)),
            pl.BlockSpec((n_e, e_dim), lambda i: (0, 0)),
        ],
        out_specs=pl.BlockSpec((1, 1, tok_blk), lambda i: (i, 0, 0)),
        out_shape=jax.ShapeDtypeStruct((n_tok // tok_blk, 1, tok_blk), jnp.int32),
    )(zfn_b, emb_nb)

    idx = idx3.reshape(n_tok)
    z_q_flat = _sc_gather(n_tok, n_e, latent)(
        embw, idx.reshape(n_tok // _IDX_CHUNK, _IDX_CHUNK))
    return z_q_flat.reshape(z.shape), idx
